# baseline (speedup 1.0000x reference)
import functools

import jax
import jax.numpy as jnp
import numpy as np
from jax import lax
from jax.experimental import pallas as pl
from jax.experimental.pallas import tpu as pltpu
from jax.experimental.pallas import tpu_sc as plsc

N = 10000
E = 320000
D_IN = 128
D_H = 128
DH2 = 64
D_OUT = 40
D2P = 48
DT = 64

NC = 2
NS = 16
NW = NC * NS
K = 128
CPW = 80
EP = NW * CPW * K
NPAD = 10112
NPT = NPAD // NS
ZR = 158
NB = 4

_MESH = plsc.VectorSubcoreMesh(
    core_axis_name="c", subcore_axis_name="s", num_cores=NC, num_subcores=NS
)
_SC_PARAMS = pltpu.CompilerParams(
    use_tc_tiling_on_sc=False, needs_layout_passes=False
)

_PAD_DST = np.asarray(N + (np.arange(EP - E) % (NPAD - N)), np.int32)
_PAD_SRC = np.asarray((np.arange(EP - E) * 37) % N, np.int32)


def _pack_perm(cols_in):
    p = np.zeros((cols_in, DT), np.float32)
    for m in range(DT):
        g, k = divmod(m, 32)
        src = 32 * g + (k // 2 if k % 2 == 0 else 16 + k // 2)
        if src < cols_in:
            p[src, m] = 1.0
    return p


_P64 = _pack_perm(DH2)
_P48 = _pack_perm(D2P)


def _fill(ref, nrows, ncols, value):
    val = jnp.full((16,), value, jnp.float32)

    def body(i, _):
        for j in range(ncols // 16):
            ref[i, pl.ds(j * 16, 16)] = val
        return 0

    lax.fori_loop(0, nrows, body, 0)


def _zero_acc(acc, zbuf, s):
    for r in range(NPT // ZR):
        pltpu.sync_copy(zbuf, acc.at[pl.ds(s * NPT + r * ZR, ZR)])


def _widen(b16, f32buf, d_acc):
    mask = jnp.int32(-65536)

    def row(r, _):
        w0 = plsc.bitcast(b16[r, pl.ds(0, 32)], jnp.int32)
        f32buf[r, pl.ds(0, 16)] = plsc.bitcast(
            jnp.left_shift(w0, 16), jnp.float32)
        f32buf[r, pl.ds(16, 16)] = plsc.bitcast(
            jnp.bitwise_and(w0, mask), jnp.float32)
        w1 = plsc.bitcast(b16[r, pl.ds(32, 32)], jnp.int32)
        f32buf[r, pl.ds(32, 16)] = plsc.bitcast(
            jnp.left_shift(w1, 16), jnp.float32)
        if d_acc == 64:
            f32buf[r, pl.ds(48, 16)] = plsc.bitcast(
                jnp.bitwise_and(w1, mask), jnp.float32)
        return 0

    lax.fori_loop(0, K, row, 0)


def _scatter_pass(y_hbm, idxs, idxd, b16s, f32s, acc, d_acc, sgs, sss):
    def body(t, _):
        gs = [
            pltpu.async_copy(y_hbm.at[idxs.at[NB * t + u]], b16s[u], sgs[u])
            for u in range(NB)
        ]
        ss = [None, None]
        for u in range(NB):
            p = u % 2
            gs[u].wait()
            if u >= 2:
                ss[p].wait()
            _widen(b16s[u], f32s[p], d_acc)
            ss[p] = pltpu.async_copy(
                f32s[p], acc.at[idxd.at[NB * t + u]], sss[p], add=True)
        ss[0].wait()
        ss[1].wait()
        return 0

    lax.fori_loop(0, CPW // NB, body, 0)


@functools.partial(
    pl.kernel,
    out_type=jax.ShapeDtypeStruct((NC * NPAD, 16), jnp.float32),
    mesh=_MESH,
    scratch_types=[
        pltpu.VMEM((CPW, K), jnp.int32),
        pltpu.VMEM((K, 16), jnp.float32),
        pltpu.VMEM((ZR, 16), jnp.float32),
        pltpu.VMEM_SHARED((NPAD, 16), jnp.float32),
        pltpu.SemaphoreType.DMA,
    ],
    compiler_params=_SC_PARAMS,
)
def _deg_kernel(dst_hbm, out_hbm, idxd, ones_v, zbuf, acc, sem):
    c = lax.axis_index("c")
    s = lax.axis_index("s")
    wid = c * NS + s
    _fill(ones_v, K, 16, 1.0)
    _fill(zbuf, ZR, 16, 0.0)
    _zero_acc(acc, zbuf, s)
    plsc.subcore_barrier()
    pltpu.sync_copy(dst_hbm.at[pl.ds(wid * CPW, CPW)], idxd)

    def body(t, _):
        ds_ = [
            pltpu.async_copy(ones_v, acc.at[idxd.at[4 * t + u]], sem, add=True)
            for u in range(4)
        ]
        for d_ in ds_:
            d_.wait()
        return 0

    lax.fori_loop(0, CPW // 4, body, 0)
    plsc.subcore_barrier()
    pltpu.sync_copy(
        acc.at[pl.ds(s * NPT, NPT)], out_hbm.at[pl.ds(c * NPAD + s * NPT, NPT)]
    )


@functools.partial(
    pl.kernel,
    out_type=[
        jax.ShapeDtypeStruct((NC * NPAD, DH2), jnp.float32),
        jax.ShapeDtypeStruct((NC * NPAD, DH2), jnp.float32),
    ],
    mesh=_MESH,
    scratch_types=[
        pltpu.VMEM((CPW, K), jnp.int32),
        pltpu.VMEM((CPW, K), jnp.int32),
        pltpu.VMEM((K, DT), jnp.bfloat16),
        pltpu.VMEM((K, DT), jnp.bfloat16),
        pltpu.VMEM((K, DT), jnp.bfloat16),
        pltpu.VMEM((K, DT), jnp.bfloat16),
        pltpu.VMEM((K, DH2), jnp.float32),
        pltpu.VMEM((K, DH2), jnp.float32),
        pltpu.VMEM((ZR, DH2), jnp.float32),
        pltpu.VMEM_SHARED((NPAD, DH2), jnp.float32),
    ] + [pltpu.SemaphoreType.DMA] * 6,
    compiler_params=_SC_PARAMS,
)
def _prop1(ya_hbm, yb_hbm, src_hbm, dst_hbm, outa_hbm, outb_hbm, idxs, idxd,
           b0, b1, b2, b3, f0, f1, zbuf, acc, *sems):
    c = lax.axis_index("c")
    s = lax.axis_index("s")
    wid = c * NS + s
    _fill(zbuf, ZR, DH2, 0.0)
    pltpu.sync_copy(src_hbm.at[pl.ds(wid * CPW, CPW)], idxs)
    pltpu.sync_copy(dst_hbm.at[pl.ds(wid * CPW, CPW)], idxd)
    for y_hbm, out_hbm in ((ya_hbm, outa_hbm), (yb_hbm, outb_hbm)):
        _zero_acc(acc, zbuf, s)
        plsc.subcore_barrier()
        _scatter_pass(y_hbm, idxs, idxd, (b0, b1, b2, b3), (f0, f1), acc,
                      DH2, sems[:4], sems[4:])
        plsc.subcore_barrier()
        pltpu.sync_copy(
            acc.at[pl.ds(s * NPT, NPT)],
            out_hbm.at[pl.ds(c * NPAD + s * NPT, NPT)],
        )
        plsc.subcore_barrier()


@functools.partial(
    pl.kernel,
    out_type=jax.ShapeDtypeStruct((NC * NPAD, D2P), jnp.float32),
    mesh=_MESH,
    scratch_types=[
        pltpu.VMEM((CPW, K), jnp.int32),
        pltpu.VMEM((CPW, K), jnp.int32),
        pltpu.VMEM((K, DT), jnp.bfloat16),
        pltpu.VMEM((K, DT), jnp.bfloat16),
        pltpu.VMEM((K, DT), jnp.bfloat16),
        pltpu.VMEM((K, DT), jnp.bfloat16),
        pltpu.VMEM((K, D2P), jnp.float32),
        pltpu.VMEM((K, D2P), jnp.float32),
        pltpu.VMEM((ZR, D2P), jnp.float32),
        pltpu.VMEM_SHARED((NPAD, D2P), jnp.float32),
    ] + [pltpu.SemaphoreType.DMA] * 6,
    compiler_params=_SC_PARAMS,
)
def _prop2(y_hbm, src_hbm, dst_hbm, out_hbm, idxs, idxd, b0, b1, b2, b3, f0,
           f1, zbuf, acc, *sems):
    c = lax.axis_index("c")
    s = lax.axis_index("s")
    wid = c * NS + s
    _fill(zbuf, ZR, D2P, 0.0)
    _zero_acc(acc, zbuf, s)
    plsc.subcore_barrier()
    pltpu.sync_copy(src_hbm.at[pl.ds(wid * CPW, CPW)], idxs)
    pltpu.sync_copy(dst_hbm.at[pl.ds(wid * CPW, CPW)], idxd)
    _scatter_pass(y_hbm, idxs, idxd, (b0, b1, b2, b3), (f0, f1), acc, D2P,
                  sems[:4], sems[4:])
    plsc.subcore_barrier()
    pltpu.sync_copy(
        acc.at[pl.ds(s * NPT, NPT)], out_hbm.at[pl.ds(c * NPAD + s * NPT, NPT)]
    )


R = 1264
GRID = NPAD // R


def _tc1_body(dega_ref, degb_ref, x_ref, w1_ref, p64_ref, ya_ref, yb_ref,
              dinv_ref):
    deg = dega_ref[:, 0:1] + degb_ref[:, 0:1] + 1.0
    dinv = lax.rsqrt(deg)
    xw = jnp.dot(x_ref[...], w1_ref[...], preferred_element_type=jnp.float32)
    y1 = xw * dinv
    p64 = p64_ref[...]
    ya_ref[...] = jnp.dot(y1[:, :DH2], p64,
                          preferred_element_type=jnp.float32).astype(jnp.bfloat16)
    yb_ref[...] = jnp.dot(y1[:, DH2:], p64,
                          preferred_element_type=jnp.float32).astype(jnp.bfloat16)
    dinv_ref[...] = jnp.broadcast_to(dinv, (R, 8))


def _tc1(degp, xpad, w1, p64):
    return pl.pallas_call(
        _tc1_body,
        grid=(GRID,),
        in_specs=[
            pl.BlockSpec((R, 16), lambda i: (i, 0)),
            pl.BlockSpec((R, 16), lambda i: (i + GRID, 0)),
            pl.BlockSpec((R, D_IN), lambda i: (i, 0)),
            pl.BlockSpec((D_IN, D_H), lambda i: (0, 0)),
            pl.BlockSpec((DH2, DT), lambda i: (0, 0)),
        ],
        out_specs=[
            pl.BlockSpec((R, DT), lambda i: (i, 0)),
            pl.BlockSpec((R, DT), lambda i: (i, 0)),
            pl.BlockSpec((R, 8), lambda i: (i, 0)),
        ],
        out_shape=[
            jax.ShapeDtypeStruct((NPAD, DT), jnp.bfloat16),
            jax.ShapeDtypeStruct((NPAD, DT), jnp.bfloat16),
            jax.ShapeDtypeStruct((NPAD, 8), jnp.float32),
        ],
    )(degp, degp, xpad, w1, p64)


def _tc2_body(za_a, za_b, zb_a, zb_b, ya_ref, yb_ref, dinv_ref, b1_ref,
              w2_ref, p64t_ref, p48_ref, y2_ref):
    dinv = dinv_ref[:, 0:1]
    p64t = p64t_ref[...]
    ya = jnp.dot(ya_ref[...].astype(jnp.float32), p64t,
                 preferred_element_type=jnp.float32)
    yb = jnp.dot(yb_ref[...].astype(jnp.float32), p64t,
                 preferred_element_type=jnp.float32)
    zlo = za_a[...] + za_b[...] + ya
    zhi = zb_a[...] + zb_b[...] + yb
    z = jnp.concatenate([zlo, zhi], axis=1)
    h = jnp.maximum(z * dinv + b1_ref[...], 0.0)
    y2 = jnp.dot(h, w2_ref[...], preferred_element_type=jnp.float32) * dinv
    y2_ref[...] = jnp.dot(y2, p48_ref[...],
                          preferred_element_type=jnp.float32).astype(jnp.bfloat16)


def _tc2(acca, accb, ya, yb, dinv, b1, w2p, p64t, p48):
    halfa = pl.BlockSpec((R, DH2), lambda i: (i, 0))
    halfb = pl.BlockSpec((R, DH2), lambda i: (i + GRID, 0))
    packed = pl.BlockSpec((R, DT), lambda i: (i, 0))
    return pl.pallas_call(
        _tc2_body,
        grid=(GRID,),
        in_specs=[
            halfa, halfb, halfa, halfb, packed, packed,
            pl.BlockSpec((R, 8), lambda i: (i, 0)),
            pl.BlockSpec((1, D_H), lambda i: (0, 0)),
            pl.BlockSpec((D_H, D2P), lambda i: (0, 0)),
            pl.BlockSpec((DT, DH2), lambda i: (0, 0)),
            pl.BlockSpec((D2P, DT), lambda i: (0, 0)),
        ],
        out_specs=pl.BlockSpec((R, DT), lambda i: (i, 0)),
        out_shape=jax.ShapeDtypeStruct((NPAD, DT), jnp.bfloat16),
    )(acca, acca, accb, accb, ya, yb, dinv, b1, w2p, p64t, p48)


def _tc3_body(acca_ref, accb_ref, y2_ref, dinv_ref, b2_ref, p48t_ref, out_ref):
    dinv = dinv_ref[:, 0:1]
    y2 = jnp.dot(y2_ref[...].astype(jnp.float32), p48t_ref[...],
                 preferred_element_type=jnp.float32)
    z = (acca_ref[...] + accb_ref[...] + y2) * dinv
    o = z[:, :D_OUT] + b2_ref[...]
    m = jnp.max(o, axis=1, keepdims=True)
    e = jnp.exp(o - m)
    out_ref[...] = e / jnp.sum(e, axis=1, keepdims=True)


def _tc3(acc2, y2, dinv, b2, p48t):
    return pl.pallas_call(
        _tc3_body,
        grid=(GRID,),
        in_specs=[
            pl.BlockSpec((R, D2P), lambda i: (i, 0)),
            pl.BlockSpec((R, D2P), lambda i: (i + GRID, 0)),
            pl.BlockSpec((R, DT), lambda i: (i, 0)),
            pl.BlockSpec((R, 8), lambda i: (i, 0)),
            pl.BlockSpec((1, D_OUT), lambda i: (0, 0)),
            pl.BlockSpec((DT, D2P), lambda i: (0, 0)),
        ],
        out_specs=pl.BlockSpec((R, D_OUT), lambda i: (i, 0)),
        out_shape=jax.ShapeDtypeStruct((NPAD, D_OUT), jnp.float32),
    )(acc2, acc2, y2, dinv, b2, p48t)


def kernel(x, edge_index, W1, b1, W2, b2):
    src = jnp.concatenate([edge_index[0], jnp.asarray(_PAD_SRC)])
    dst = jnp.concatenate([edge_index[1], jnp.asarray(_PAD_DST)])
    src2 = src.reshape(NW * CPW, K)
    dst2 = dst.reshape(NW * CPW, K)
    w2p = jnp.pad(W2, ((0, 0), (0, D2P - D_OUT)))
    b1r = b1.reshape(1, D_H)
    b2r = b2.reshape(1, D_OUT)
    p64 = jnp.asarray(_P64)
    p48 = jnp.asarray(_P48)

    xpad = jnp.pad(x, ((0, NPAD - N), (0, 0)))
    degp = _deg_kernel(dst2)
    ya16, yb16, dinv = _tc1(degp, xpad, W1, p64)
    acc1a, acc1b = _prop1(ya16, yb16, src2, dst2)
    y216 = _tc2(acc1a, acc1b, ya16, yb16, dinv, b1r, w2p, p64.T, p48)
    acc2 = _prop2(y216, src2, dst2)
    return _tc3(acc2, y216, dinv, b2r, p48.T)[:N]
